# baseline (speedup 1.0000x reference)
import functools

import jax
import jax.numpy as jnp
from jax import lax
from jax.experimental import pallas as pl
from jax.experimental.pallas import tpu as pltpu
from jax.experimental.pallas import tpu_sc as plsc

N = 10000
E = 320000
D = 128
H = 128
OUT = 16
EPS = 1e-05
SLOPE = 0.01

NC = 2
NS = 16
NW = NC * NS
EPW = E // NW
CHUNK = 400
NCHUNK = EPW // CHUNK
GROUPS = CHUNK // 16
EC = 16
NCH = EPW // EC
NBUF = 5
RPT = 624
TAIL = N - NS * RPT

_SC_MESH = plsc.VectorSubcoreMesh(core_axis_name="c", subcore_axis_name="s",
                                  num_cores=NC, num_subcores=NS)
_SC_PARAMS = pltpu.CompilerParams(needs_layout_passes=False)


@functools.partial(
    pl.kernel,
    out_type=jax.ShapeDtypeStruct((NW * 2 * N,), jnp.float32),
    mesh=_SC_MESH,
    compiler_params=_SC_PARAMS,
    scratch_types=[
        pltpu.VMEM((2 * N,), jnp.float32),
        pltpu.VMEM((CHUNK,), jnp.int32),
        pltpu.VMEM((CHUNK,), jnp.int32),
    ],
)
def _deg_kernel(src_hbm, dst_hbm, out_hbm, acc, srcv, dstv):
    c = lax.axis_index("c")
    s = lax.axis_index("s")
    wid = c * NS + s
    zeros16 = jnp.zeros((16,), jnp.float32)

    def zero_body(i, _):
        acc[pl.ds(i * 16, 16)] = zeros16
        return 0

    lax.fori_loop(0, 2 * N // 16, zero_body, 0)

    ones16 = jnp.ones((16,), jnp.float32)
    offN = jnp.full((16,), N, jnp.int32)

    def chunk_body(i, _):
        base = wid * EPW + i * CHUNK
        pltpu.sync_copy(src_hbm.at[pl.ds(base, CHUNK)], srcv)
        pltpu.sync_copy(dst_hbm.at[pl.ds(base, CHUNK)], dstv)

        def group_body(g, _):
            sv = srcv[pl.ds(g * 16, 16)]
            dv = dstv[pl.ds(g * 16, 16)]
            plsc.addupdate_scatter(acc, [sv], ones16)
            plsc.addupdate_scatter(acc, [dv + offN], ones16)
            return 0

        lax.fori_loop(0, GROUPS, group_body, 0)
        return 0

    lax.fori_loop(0, NCHUNK, chunk_body, 0)
    pltpu.sync_copy(acc, out_hbm.at[pl.ds(wid * 2 * N, 2 * N)])


@functools.partial(
    pl.kernel,
    out_type=jax.ShapeDtypeStruct((NC * N, H), jnp.float32),
    mesh=_SC_MESH,
    compiler_params=_SC_PARAMS,
    scratch_types=[
        pltpu.VMEM_SHARED((N, H), jnp.float32),
        pltpu.VMEM((EC, H), jnp.float32),
        pltpu.VMEM((EC, H), jnp.float32),
        pltpu.VMEM((EC, H), jnp.float32),
        pltpu.VMEM((EC, H), jnp.float32),
        pltpu.VMEM((EC, H), jnp.float32),
        pltpu.VMEM((EPW,), jnp.int32),
        pltpu.VMEM((EPW,), jnp.int32),
        pltpu.VMEM((EPW,), jnp.float32),
        pltpu.SemaphoreType.DMA,
        pltpu.SemaphoreType.DMA,
        pltpu.SemaphoreType.DMA,
        pltpu.SemaphoreType.DMA,
        pltpu.SemaphoreType.DMA,
        pltpu.SemaphoreType.DMA,
        pltpu.SemaphoreType.DMA,
        pltpu.SemaphoreType.DMA,
        pltpu.SemaphoreType.DMA,
        pltpu.SemaphoreType.DMA,
    ],
)
def _edge_kernel(z_hbm, src_hbm, dst_hbm, ew_hbm, zinit_hbm, out_hbm,
                 acc, rows0, rows1, rows2, rows3, rows4, srcv, dstv, ewb,
                 semg0, semg1, semg2, semg3, semg4,
                 sems0, sems1, sems2, sems3, sems4):
    c = lax.axis_index("c")
    s = lax.axis_index("s")
    wid = c * NS + s
    base = wid * EPW

    pltpu.sync_copy(zinit_hbm, acc.at[pl.ds(s * RPT, RPT)])

    @pl.when(s == NS - 1)
    def _():
        pltpu.sync_copy(zinit_hbm.at[pl.ds(0, TAIL)],
                        acc.at[pl.ds(NS * RPT, TAIL)])

    plsc.subcore_barrier()

    pltpu.sync_copy(src_hbm.at[pl.ds(base, EPW)], srcv)
    pltpu.sync_copy(dst_hbm.at[pl.ds(base, EPW)], dstv)
    pltpu.sync_copy(ew_hbm.at[pl.ds(base, EPW)], ewb)

    bufs = ((rows0, semg0, sems0), (rows1, semg1, sems1),
            (rows2, semg2, sems2), (rows3, semg3, sems3),
            (rows4, semg4, sems4))

    pltpu.async_copy(z_hbm.at[srcv.at[pl.ds(0, EC)]], rows0, semg0)
    pltpu.async_copy(z_hbm.at[srcv.at[pl.ds(EC, EC)]], rows1, semg1)
    pltpu.async_copy(z_hbm.at[srcv.at[pl.ds(2 * EC, EC)]], rows2, semg2)

    def chunk(j, bi, wait_scat, issue_gather):
        rows, semg, sems = bufs[bi]
        rows_t, semg_t, sems_t = bufs[(bi + 3) % NBUF]
        ebase = j * EC
        pltpu.make_async_copy(z_hbm.at[srcv.at[pl.ds(ebase, EC)]],
                              rows, semg).wait()
        if wait_scat:
            pltpu.make_async_copy(
                rows_t, acc.at[dstv.at[pl.ds((j - 2) * EC, EC)]],
                sems_t).wait()
        if issue_gather:
            pltpu.async_copy(z_hbm.at[srcv.at[pl.ds((j + 3) * EC, EC)]],
                             rows_t, semg_t)

        @plsc.parallel_loop(0, EC, unroll=8)
        def _mul(e):
            w = plsc.load_gather(ewb, [jnp.full((16,), ebase + e, jnp.int32)])
            for k in range(H // 16):
                sl = pl.ds(k * 16, 16)
                rows[e, sl] = rows[e, sl] * w

        pltpu.async_copy(rows, acc.at[dstv.at[pl.ds(ebase, EC)]], sems,
                         add=True)

    chunk(0, 0, False, True)
    chunk(1, 1, False, True)
    chunk(2, 2, True, True)
    chunk(3, 3, True, True)
    chunk(4, 4, True, True)

    def quint_body(q, _):
        j = NBUF * q
        chunk(j, 0, True, True)
        chunk(j + 1, 1, True, True)
        chunk(j + 2, 2, True, True)
        chunk(j + 3, 3, True, True)
        chunk(j + 4, 4, True, True)
        return 0

    lax.fori_loop(1, NCH // NBUF - 1, quint_body, 0)

    chunk(NCH - 5, 0, True, True)
    chunk(NCH - 4, 1, True, True)
    chunk(NCH - 3, 2, True, False)
    chunk(NCH - 2, 3, True, False)
    chunk(NCH - 1, 4, True, False)

    pltpu.make_async_copy(
        rows3, acc.at[dstv.at[pl.ds((NCH - 2) * EC, EC)]], sems3).wait()
    pltpu.make_async_copy(
        rows4, acc.at[dstv.at[pl.ds((NCH - 1) * EC, EC)]], sems4).wait()
    plsc.subcore_barrier()
    pltpu.sync_copy(acc.at[pl.ds(s * RPT, RPT)],
                    out_hbm.at[pl.ds(c * N + s * RPT, RPT)])

    @pl.when(s == NS - 1)
    def _():
        pltpu.sync_copy(acc.at[pl.ds(NS * RPT, TAIL)],
                        out_hbm.at[pl.ds(c * N + NS * RPT, TAIL)])


_BLK = 1000
_GRID = N // _BLK


def _degsum_body(p_ref, o_ref):
    deg = jnp.sum(p_ref[...], axis=0)
    o_ref[...] = lax.rsqrt(jnp.clip(deg, 1.0, None))


def _deg_scales(parts):
    return pl.pallas_call(
        _degsum_body,
        in_specs=[pl.BlockSpec((NW, 2, N), lambda: (0, 0, 0))],
        out_specs=pl.BlockSpec((2, N), lambda: (0, 0)),
        out_shape=jax.ShapeDtypeStruct((2, N), jnp.float32),
    )(parts)


def _mm_body(x_ref, s_ref, w_ref, o_ref):
    o_ref[...] = jnp.dot(x_ref[...] * s_ref[...], w_ref[...],
                         preferred_element_type=jnp.float32)


def _scaled_matmul(x, s_col, w):
    return pl.pallas_call(
        _mm_body,
        grid=(_GRID,),
        in_specs=[
            pl.BlockSpec((_BLK, D), lambda i: (i, 0)),
            pl.BlockSpec((_BLK, 1), lambda i: (i, 0)),
            pl.BlockSpec((D, H), lambda i: (0, 0)),
        ],
        out_specs=pl.BlockSpec((_BLK, H), lambda i: (i, 0)),
        out_shape=jax.ShapeDtypeStruct((N, H), jnp.float32),
    )(x, s_col, w)


def _stats_body(p_ref, s_ref, h_ref, sums_ref):
    h = (p_ref[0] + p_ref[1]) * s_ref[...]
    h_ref[...] = h

    @pl.when(pl.program_id(0) == 0)
    def _():
        sums_ref[...] = jnp.zeros_like(sums_ref)

    sums_ref[...] += jnp.stack(
        (jnp.sum(h, axis=0), jnp.sum(h * h, axis=0)))


def _combine_stats(parts, s_col):
    return pl.pallas_call(
        _stats_body,
        grid=(_GRID,),
        in_specs=[
            pl.BlockSpec((2, _BLK, H), lambda i: (0, i, 0)),
            pl.BlockSpec((_BLK, 1), lambda i: (i, 0)),
        ],
        out_specs=[
            pl.BlockSpec((_BLK, H), lambda i: (i, 0)),
            pl.BlockSpec((2, H), lambda i: (0, 0)),
        ],
        out_shape=[
            jax.ShapeDtypeStruct((N, H), jnp.float32),
            jax.ShapeDtypeStruct((2, H), jnp.float32),
        ],
    )(parts, s_col)


def _affine(stats_ref, gam_ref, bet_ref, alp_ref):
    mean = stats_ref[0:1] * (1.0 / N)
    e2 = stats_ref[1:2] * (1.0 / N)
    am = alp_ref[...] * mean
    var = e2 - 2.0 * am * mean + am * am
    scale = gam_ref[...] * lax.rsqrt(var + EPS)
    shift = bet_ref[...] - scale * am
    return scale, shift


def _leaky(x):
    return jnp.where(x >= 0, x, SLOPE * x)


def _norm_mm_body(h_ref, stats_ref, gam_ref, bet_ref, alp_ref, s_ref, w_ref,
                  z_ref, rsum_ref):
    scale, shift = _affine(stats_ref, gam_ref, bet_ref, alp_ref)
    h = _leaky(h_ref[...] * scale + shift)

    @pl.when(pl.program_id(0) == 0)
    def _():
        rsum_ref[...] = jnp.zeros_like(rsum_ref)

    rsum_ref[...] += jnp.sum(h, axis=0, keepdims=True)
    z_ref[...] = jnp.dot(h * s_ref[...], w_ref[...],
                         preferred_element_type=jnp.float32)


def _norm_matmul(h_pre, stats, gam, bet, alp, s_col, w):
    return pl.pallas_call(
        _norm_mm_body,
        grid=(_GRID,),
        in_specs=[
            pl.BlockSpec((_BLK, H), lambda i: (i, 0)),
            pl.BlockSpec((2, H), lambda i: (0, 0)),
            pl.BlockSpec((1, H), lambda i: (0, 0)),
            pl.BlockSpec((1, H), lambda i: (0, 0)),
            pl.BlockSpec((1, H), lambda i: (0, 0)),
            pl.BlockSpec((_BLK, 1), lambda i: (i, 0)),
            pl.BlockSpec((H, H), lambda i: (0, 0)),
        ],
        out_specs=[
            pl.BlockSpec((_BLK, H), lambda i: (i, 0)),
            pl.BlockSpec((1, H), lambda i: (0, 0)),
        ],
        out_shape=[
            jax.ShapeDtypeStruct((N, H), jnp.float32),
            jax.ShapeDtypeStruct((1, H), jnp.float32),
        ],
    )(h_pre, stats, gam, bet, alp, s_col, w)


def _final_body(h_ref, stats_ref, gam_ref, bet_ref, alp_ref, r1_ref, wc_ref,
                o_ref, acc):
    scale, shift = _affine(stats_ref, gam_ref, bet_ref, alp_ref)
    h = _leaky(h_ref[...] * scale + shift)

    @pl.when(pl.program_id(0) == 0)
    def _():
        acc[...] = jnp.zeros_like(acc)

    acc[...] += jnp.sum(h, axis=0, keepdims=True)

    @pl.when(pl.program_id(0) == _GRID - 1)
    def _():
        r = jnp.concatenate((r1_ref[...], acc[...]), axis=1) * (1.0 / N)
        o_ref[...] = lax.dot_general(
            r, wc_ref[...], (((1,), (1,)), ((), ())),
            preferred_element_type=jnp.float32)


def _final(h_pre, stats, gam, bet, alp, r1sum, wc):
    return pl.pallas_call(
        _final_body,
        grid=(_GRID,),
        in_specs=[
            pl.BlockSpec((_BLK, H), lambda i: (i, 0)),
            pl.BlockSpec((2, H), lambda i: (0, 0)),
            pl.BlockSpec((1, H), lambda i: (0, 0)),
            pl.BlockSpec((1, H), lambda i: (0, 0)),
            pl.BlockSpec((1, H), lambda i: (0, 0)),
            pl.BlockSpec((1, H), lambda i: (0, 0)),
            pl.BlockSpec((OUT, 2 * H), lambda i: (0, 0)),
        ],
        out_specs=pl.BlockSpec((1, OUT), lambda i: (0, 0)),
        out_shape=jax.ShapeDtypeStruct((1, OUT), jnp.float32),
        scratch_shapes=[pltpu.VMEM((1, H), jnp.float32)],
    )(h_pre, stats, gam, bet, alp, r1sum, wc)


def kernel(features, edge_index, edge_weights, W1, W2, Wc,
           gamma1, beta1, alpha1, gamma2, beta2, alpha2):
    src = edge_index[0]
    dst = edge_index[1]
    zinit = jnp.zeros((RPT, H), jnp.float32)

    degp = _deg_kernel(src, dst).reshape(NW, 2, N)
    sres = _deg_scales(degp)
    s_out = sres[0].reshape(N, 1)
    s_in = sres[1].reshape(N, 1)

    g1 = gamma1.reshape(1, H)
    b1 = beta1.reshape(1, H)
    a1 = alpha1.reshape(1, H)
    g2 = gamma2.reshape(1, H)
    b2 = beta2.reshape(1, H)
    a2 = alpha2.reshape(1, H)

    z1 = _scaled_matmul(features, s_out, W1)
    parts1 = _edge_kernel(z1, src, dst, edge_weights, zinit).reshape(NC, N, H)
    h1_pre, stats1 = _combine_stats(parts1, s_in)

    z2, r1sum = _norm_matmul(h1_pre, stats1, g1, b1, a1, s_out, W2)
    parts2 = _edge_kernel(z2, src, dst, edge_weights, zinit).reshape(NC, N, H)
    h2_pre, stats2 = _combine_stats(parts2, s_in)

    return _final(h2_pre, stats2, g2, b2, a2, r1sum, Wc)


# speedup vs baseline: 1.2622x; 1.2622x over previous
import functools

import jax
import jax.numpy as jnp
from jax import lax
from jax.experimental import pallas as pl
from jax.experimental.pallas import tpu as pltpu
from jax.experimental.pallas import tpu_sc as plsc

N = 10000
E = 320000
D = 128
H = 128
OUT = 16
EPS = 1e-05
SLOPE = 0.01

NC = 2
NS = 16
NW = NC * NS
EPW = E // NW
CHUNK = 400
NCHUNK = EPW // CHUNK
GROUPS = CHUNK // 16
EC = 40
NCH = EPW // EC
RPT = 624
TAIL = N - NS * RPT

_SC_MESH = plsc.VectorSubcoreMesh(core_axis_name="c", subcore_axis_name="s",
                                  num_cores=NC, num_subcores=NS)
_SC_PARAMS = pltpu.CompilerParams(needs_layout_passes=False)


@functools.partial(
    pl.kernel,
    out_type=jax.ShapeDtypeStruct((NW * 2 * N,), jnp.float32),
    mesh=_SC_MESH,
    compiler_params=_SC_PARAMS,
    scratch_types=[
        pltpu.VMEM((2 * N,), jnp.float32),
        pltpu.VMEM((CHUNK,), jnp.int32),
        pltpu.VMEM((CHUNK,), jnp.int32),
    ],
)
def _deg_kernel(src_hbm, dst_hbm, out_hbm, acc, srcv, dstv):
    c = lax.axis_index("c")
    s = lax.axis_index("s")
    wid = c * NS + s
    zeros16 = jnp.zeros((16,), jnp.float32)

    def zero_body(i, _):
        acc[pl.ds(i * 16, 16)] = zeros16
        return 0

    lax.fori_loop(0, 2 * N // 16, zero_body, 0)

    ones16 = jnp.ones((16,), jnp.float32)
    offN = jnp.full((16,), N, jnp.int32)

    def chunk_body(i, _):
        base = wid * EPW + i * CHUNK
        pltpu.sync_copy(src_hbm.at[pl.ds(base, CHUNK)], srcv)
        pltpu.sync_copy(dst_hbm.at[pl.ds(base, CHUNK)], dstv)

        def group_body(g, _):
            sv = srcv[pl.ds(g * 16, 16)]
            dv = dstv[pl.ds(g * 16, 16)]
            plsc.addupdate_scatter(acc, [sv], ones16)
            plsc.addupdate_scatter(acc, [dv + offN], ones16)
            return 0

        lax.fori_loop(0, GROUPS, group_body, 0)
        return 0

    lax.fori_loop(0, NCHUNK, chunk_body, 0)
    pltpu.sync_copy(acc, out_hbm.at[pl.ds(wid * 2 * N, 2 * N)])


@functools.partial(
    pl.kernel,
    out_type=jax.ShapeDtypeStruct((NC * N, H), jnp.float32),
    mesh=_SC_MESH,
    compiler_params=_SC_PARAMS,
    scratch_types=[
        pltpu.VMEM_SHARED((N, H), jnp.float32),
        pltpu.VMEM((EC, H), jnp.float32),
        pltpu.VMEM((EC, H), jnp.float32),
        pltpu.VMEM((EC, H), jnp.float32),
        pltpu.VMEM((EC, H), jnp.float32),
        pltpu.VMEM((EPW,), jnp.int32),
        pltpu.VMEM((EPW,), jnp.int32),
        pltpu.VMEM((EPW,), jnp.float32),
        pltpu.SemaphoreType.DMA,
        pltpu.SemaphoreType.DMA,
        pltpu.SemaphoreType.DMA,
        pltpu.SemaphoreType.DMA,
        pltpu.SemaphoreType.DMA,
        pltpu.SemaphoreType.DMA,
        pltpu.SemaphoreType.DMA,
        pltpu.SemaphoreType.DMA,
    ],
)
def _edge_kernel(z_hbm, src_hbm, dst_hbm, ew_hbm, zinit_hbm, out_hbm,
                 acc, rows0, rows1, rows2, rows3, srcv, dstv, ewb,
                 semg0, semg1, semg2, semg3, sems0, sems1, sems2, sems3):
    c = lax.axis_index("c")
    s = lax.axis_index("s")
    wid = c * NS + s
    base = wid * EPW

    pltpu.sync_copy(zinit_hbm, acc.at[pl.ds(s * RPT, RPT)])

    @pl.when(s == NS - 1)
    def _():
        pltpu.sync_copy(zinit_hbm.at[pl.ds(0, TAIL)],
                        acc.at[pl.ds(NS * RPT, TAIL)])

    plsc.subcore_barrier()

    pltpu.sync_copy(src_hbm.at[pl.ds(base, EPW)], srcv)
    pltpu.sync_copy(dst_hbm.at[pl.ds(base, EPW)], dstv)
    pltpu.sync_copy(ew_hbm.at[pl.ds(base, EPW)], ewb)

    bufs = ((rows0, semg0, sems0), (rows1, semg1, sems1),
            (rows2, semg2, sems2), (rows3, semg3, sems3))

    pltpu.async_copy(z_hbm.at[srcv.at[pl.ds(0, EC)]], rows0, semg0)
    pltpu.async_copy(z_hbm.at[srcv.at[pl.ds(EC, EC)]], rows1, semg1)

    def chunk(j, bi, wait_scat, issue_gather):
        rows, semg, sems = bufs[bi]
        rows_t, semg_t, sems_t = bufs[(bi + 2) % 4]
        ebase = j * EC
        if wait_scat:
            pltpu.make_async_copy(
                rows_t, acc.at[dstv.at[pl.ds((j - 2) * EC, EC)]],
                sems_t).wait()
        if issue_gather:
            pltpu.async_copy(z_hbm.at[srcv.at[pl.ds((j + 2) * EC, EC)]],
                             rows_t, semg_t)
        pltpu.make_async_copy(z_hbm.at[srcv.at[pl.ds(ebase, EC)]],
                              rows, semg).wait()

        @plsc.parallel_loop(0, EC, unroll=8)
        def _mul(e):
            w = plsc.load_gather(ewb, [jnp.full((16,), ebase + e, jnp.int32)])
            for k in range(H // 16):
                sl = pl.ds(k * 16, 16)
                rows[e, sl] = rows[e, sl] * w

        pltpu.async_copy(rows, acc.at[dstv.at[pl.ds(ebase, EC)]], sems,
                         add=True)

    chunk(0, 0, False, True)
    chunk(1, 1, False, True)
    chunk(2, 2, True, True)
    chunk(3, 3, True, True)

    def quad_body(q, _):
        j = 4 * q
        chunk(j, 0, True, True)
        chunk(j + 1, 1, True, True)
        chunk(j + 2, 2, True, True)
        chunk(j + 3, 3, True, True)
        return 0

    lax.fori_loop(1, (NCH - 2) // 4, quad_body, 0)

    chunk(NCH - 2, 0, True, False)
    chunk(NCH - 1, 1, True, False)

    pltpu.make_async_copy(
        rows0, acc.at[dstv.at[pl.ds((NCH - 2) * EC, EC)]], sems0).wait()
    pltpu.make_async_copy(
        rows1, acc.at[dstv.at[pl.ds((NCH - 1) * EC, EC)]], sems1).wait()
    plsc.subcore_barrier()
    pltpu.sync_copy(acc.at[pl.ds(s * RPT, RPT)],
                    out_hbm.at[pl.ds(c * N + s * RPT, RPT)])

    @pl.when(s == NS - 1)
    def _():
        pltpu.sync_copy(acc.at[pl.ds(NS * RPT, TAIL)],
                        out_hbm.at[pl.ds(c * N + NS * RPT, TAIL)])


_BLK = 1000
_GRID = N // _BLK


def _degsum_body(p_ref, o_ref):
    deg = jnp.sum(p_ref[...], axis=0)
    o_ref[...] = lax.rsqrt(jnp.clip(deg, 1.0, None))


def _deg_scales(parts):
    return pl.pallas_call(
        _degsum_body,
        in_specs=[pl.BlockSpec((NW, 2, N), lambda: (0, 0, 0))],
        out_specs=pl.BlockSpec((2, N), lambda: (0, 0)),
        out_shape=jax.ShapeDtypeStruct((2, N), jnp.float32),
    )(parts)


def _mm_body(x_ref, s_ref, w_ref, o_ref):
    o_ref[...] = jnp.dot(x_ref[...] * s_ref[...], w_ref[...],
                         preferred_element_type=jnp.float32)


def _scaled_matmul(x, s_col, w):
    return pl.pallas_call(
        _mm_body,
        grid=(_GRID,),
        in_specs=[
            pl.BlockSpec((_BLK, D), lambda i: (i, 0)),
            pl.BlockSpec((_BLK, 1), lambda i: (i, 0)),
            pl.BlockSpec((D, H), lambda i: (0, 0)),
        ],
        out_specs=pl.BlockSpec((_BLK, H), lambda i: (i, 0)),
        out_shape=jax.ShapeDtypeStruct((N, H), jnp.float32),
    )(x, s_col, w)


def _stats_body(p_ref, s_ref, h_ref, sums_ref):
    h = (p_ref[0] + p_ref[1]) * s_ref[...]
    h_ref[...] = h

    @pl.when(pl.program_id(0) == 0)
    def _():
        sums_ref[...] = jnp.zeros_like(sums_ref)

    sums_ref[...] += jnp.stack(
        (jnp.sum(h, axis=0), jnp.sum(h * h, axis=0)))


def _combine_stats(parts, s_col):
    return pl.pallas_call(
        _stats_body,
        grid=(_GRID,),
        in_specs=[
            pl.BlockSpec((2, _BLK, H), lambda i: (0, i, 0)),
            pl.BlockSpec((_BLK, 1), lambda i: (i, 0)),
        ],
        out_specs=[
            pl.BlockSpec((_BLK, H), lambda i: (i, 0)),
            pl.BlockSpec((2, H), lambda i: (0, 0)),
        ],
        out_shape=[
            jax.ShapeDtypeStruct((N, H), jnp.float32),
            jax.ShapeDtypeStruct((2, H), jnp.float32),
        ],
    )(parts, s_col)


def _affine(stats_ref, gam_ref, bet_ref, alp_ref):
    mean = stats_ref[0:1] * (1.0 / N)
    e2 = stats_ref[1:2] * (1.0 / N)
    am = alp_ref[...] * mean
    var = e2 - 2.0 * am * mean + am * am
    scale = gam_ref[...] * lax.rsqrt(var + EPS)
    shift = bet_ref[...] - scale * am
    return scale, shift


def _leaky(x):
    return jnp.where(x >= 0, x, SLOPE * x)


def _norm_mm_body(h_ref, stats_ref, gam_ref, bet_ref, alp_ref, s_ref, w_ref,
                  z_ref, rsum_ref):
    scale, shift = _affine(stats_ref, gam_ref, bet_ref, alp_ref)
    h = _leaky(h_ref[...] * scale + shift)

    @pl.when(pl.program_id(0) == 0)
    def _():
        rsum_ref[...] = jnp.zeros_like(rsum_ref)

    rsum_ref[...] += jnp.sum(h, axis=0, keepdims=True)
    z_ref[...] = jnp.dot(h * s_ref[...], w_ref[...],
                         preferred_element_type=jnp.float32)


def _norm_matmul(h_pre, stats, gam, bet, alp, s_col, w):
    return pl.pallas_call(
        _norm_mm_body,
        grid=(_GRID,),
        in_specs=[
            pl.BlockSpec((_BLK, H), lambda i: (i, 0)),
            pl.BlockSpec((2, H), lambda i: (0, 0)),
            pl.BlockSpec((1, H), lambda i: (0, 0)),
            pl.BlockSpec((1, H), lambda i: (0, 0)),
            pl.BlockSpec((1, H), lambda i: (0, 0)),
            pl.BlockSpec((_BLK, 1), lambda i: (i, 0)),
            pl.BlockSpec((H, H), lambda i: (0, 0)),
        ],
        out_specs=[
            pl.BlockSpec((_BLK, H), lambda i: (i, 0)),
            pl.BlockSpec((1, H), lambda i: (0, 0)),
        ],
        out_shape=[
            jax.ShapeDtypeStruct((N, H), jnp.float32),
            jax.ShapeDtypeStruct((1, H), jnp.float32),
        ],
    )(h_pre, stats, gam, bet, alp, s_col, w)


def _final_body(h_ref, stats_ref, gam_ref, bet_ref, alp_ref, r1_ref, wc_ref,
                o_ref, acc):
    scale, shift = _affine(stats_ref, gam_ref, bet_ref, alp_ref)
    h = _leaky(h_ref[...] * scale + shift)

    @pl.when(pl.program_id(0) == 0)
    def _():
        acc[...] = jnp.zeros_like(acc)

    acc[...] += jnp.sum(h, axis=0, keepdims=True)

    @pl.when(pl.program_id(0) == _GRID - 1)
    def _():
        r = jnp.concatenate((r1_ref[...], acc[...]), axis=1) * (1.0 / N)
        o_ref[...] = lax.dot_general(
            r, wc_ref[...], (((1,), (1,)), ((), ())),
            preferred_element_type=jnp.float32)


def _final(h_pre, stats, gam, bet, alp, r1sum, wc):
    return pl.pallas_call(
        _final_body,
        grid=(_GRID,),
        in_specs=[
            pl.BlockSpec((_BLK, H), lambda i: (i, 0)),
            pl.BlockSpec((2, H), lambda i: (0, 0)),
            pl.BlockSpec((1, H), lambda i: (0, 0)),
            pl.BlockSpec((1, H), lambda i: (0, 0)),
            pl.BlockSpec((1, H), lambda i: (0, 0)),
            pl.BlockSpec((1, H), lambda i: (0, 0)),
            pl.BlockSpec((OUT, 2 * H), lambda i: (0, 0)),
        ],
        out_specs=pl.BlockSpec((1, OUT), lambda i: (0, 0)),
        out_shape=jax.ShapeDtypeStruct((1, OUT), jnp.float32),
        scratch_shapes=[pltpu.VMEM((1, H), jnp.float32)],
    )(h_pre, stats, gam, bet, alp, r1sum, wc)


def kernel(features, edge_index, edge_weights, W1, W2, Wc,
           gamma1, beta1, alpha1, gamma2, beta2, alpha2):
    src = edge_index[0]
    dst = edge_index[1]
    zinit = jnp.zeros((RPT, H), jnp.float32)

    degp = _deg_kernel(src, dst).reshape(NW, 2, N)
    sres = _deg_scales(degp)
    s_out = sres[0].reshape(N, 1)
    s_in = sres[1].reshape(N, 1)

    g1 = gamma1.reshape(1, H)
    b1 = beta1.reshape(1, H)
    a1 = alpha1.reshape(1, H)
    g2 = gamma2.reshape(1, H)
    b2 = beta2.reshape(1, H)
    a2 = alpha2.reshape(1, H)

    z1 = _scaled_matmul(features, s_out, W1)
    parts1 = _edge_kernel(z1, src, dst, edge_weights, zinit).reshape(NC, N, H)
    h1_pre, stats1 = _combine_stats(parts1, s_in)

    z2, r1sum = _norm_matmul(h1_pre, stats1, g1, b1, a1, s_out, W2)
    parts2 = _edge_kernel(z2, src, dst, edge_weights, zinit).reshape(NC, N, H)
    h2_pre, stats2 = _combine_stats(parts2, s_in)

    return _final(h2_pre, stats2, g2, b2, a2, r1sum, Wc)
